# baseline (speedup 1.0000x reference)
import functools

import jax
import jax.numpy as jnp
from jax import lax
from jax.experimental import pallas as pl
from jax.experimental.pallas import tpu as pltpu
from jax.experimental.pallas import tpu_sc as plsc

N = 10000
D = 128
DH = D // 2
H = 256
C = 40
E = 320000

NP = 10240
CP = 48
NC = 2
NS = 16
K = 128
RPT = NP // NS

CH1 = 158
CH2 = 79
EP = NS * CH1 * K

_NB = 4

_MESH = plsc.VectorSubcoreMesh(core_axis_name="c", subcore_axis_name="s")


def _zero_fill(ref, i, width):
  for t in range(width // 16):
    ref[i, pl.ds(t * 16, 16)] = jnp.zeros((16,), jnp.float32)


def _edge_pipeline(tbl, src_v, dst_v, rows, gsems, acc_sh, num_chunks,
                   ones_v=None, cnt_sh=None, ones_parity=0):
  pltpu.async_copy(tbl.at[src_v.at[0]], rows[0], gsems[0])
  pltpu.async_copy(tbl.at[src_v.at[1]], rows[1], gsems[1])

  def group(g, carry):
    j0 = g * _NB
    for b in range(_NB):
      jj = j0 + b
      bn = (b + 2) % _NB

      @pl.when(jj < num_chunks)
      def _(jj=jj, b=b, bn=bn):
        pltpu.make_async_copy(tbl.at[src_v.at[jj]], rows[b], gsems[b]).wait()
        pltpu.sync_copy(rows[b], acc_sh.at[dst_v.at[jj]], add=True)
        if ones_v is not None:
          @pl.when(jj % 2 == ones_parity)
          def _():
            pltpu.sync_copy(ones_v, cnt_sh.at[dst_v.at[jj]], add=True)

        @pl.when(jj + 2 < num_chunks)
        def _():
          pltpu.async_copy(tbl.at[src_v.at[jj + 2]], rows[bn], gsems[bn])

    return carry

  lax.fori_loop(0, (num_chunks + _NB - 1) // _NB, group, 0)


_L1_OUT = [
    jax.ShapeDtypeStruct((NP, DH), jnp.float32),
    jax.ShapeDtypeStruct((NP, DH), jnp.float32),
    jax.ShapeDtypeStruct((NC, NP, 16), jnp.float32),
]
_L1_SCRATCH = (
    [pltpu.VMEM((CH1, K), jnp.int32),
     pltpu.VMEM((CH1, K), jnp.int32)]
    + [pltpu.VMEM((K, DH), jnp.float32)] * _NB
    + [pltpu.VMEM((K, 16), jnp.float32),
       pltpu.VMEM((K, 16), jnp.float32)]
    + [pltpu.VMEM_SHARED((NP, DH), jnp.float32),
       pltpu.VMEM_SHARED((NP, 16), jnp.float32)]
    + [pltpu.SemaphoreType.DMA] * _NB
)


@functools.partial(pl.kernel, out_type=_L1_OUT, mesh=_MESH,
                   scratch_types=_L1_SCRATCH,
                   compiler_params=pltpu.CompilerParams(
                       use_tc_tiling_on_sc=False))
def _sc_layer1(x_lo, x_hi, src_hbm, dst_hbm, lo_out, hi_out, cnt_out,
               src_v, dst_v, r0, r1, r2, r3, ones_v, zcnt_v, acc_sh, cnt_sh,
               g0, g1, g2, g3):
  cid = lax.axis_index("c")
  sid = lax.axis_index("s")
  rows = (r0, r1, r2, r3)
  gsems = (g0, g1, g2, g3)

  def init_row(i, carry):
    _zero_fill(r0, i, DH)
    ones_v[i, :] = jnp.ones((16,), jnp.float32)
    zcnt_v[i, :] = jnp.zeros((16,), jnp.float32)
    return carry

  lax.fori_loop(0, K, init_row, 0)
  base = sid * RPT
  for t in range(RPT // K):
    pltpu.sync_copy(r0, acc_sh.at[pl.ds(base + t * K, K)])
    pltpu.sync_copy(zcnt_v, cnt_sh.at[pl.ds(base + t * K, K)])
  plsc.subcore_barrier()

  pltpu.sync_copy(src_hbm.at[sid], src_v)
  pltpu.sync_copy(dst_hbm.at[sid], dst_v)

  @pl.when(cid == 0)
  def _():
    _edge_pipeline(x_lo, src_v, dst_v, rows, gsems, acc_sh, CH1,
                   ones_v, cnt_sh, ones_parity=0)

  @pl.when(cid == 1)
  def _():
    _edge_pipeline(x_hi, src_v, dst_v, rows, gsems, acc_sh, CH1,
                   ones_v, cnt_sh, ones_parity=1)

  plsc.subcore_barrier()

  @pl.when(cid == 0)
  def _():
    pltpu.sync_copy(acc_sh.at[pl.ds(base, RPT)], lo_out.at[pl.ds(base, RPT)])

  @pl.when(cid == 1)
  def _():
    pltpu.sync_copy(acc_sh.at[pl.ds(base, RPT)], hi_out.at[pl.ds(base, RPT)])

  pltpu.sync_copy(cnt_sh.at[pl.ds(base, RPT)],
                  cnt_out.at[cid, pl.ds(base, RPT)])


_L2_OUT = [jax.ShapeDtypeStruct((NC, NP, CP), jnp.float32)]
_L2_SCRATCH = (
    [pltpu.VMEM((CH2, K), jnp.int32),
     pltpu.VMEM((CH2, K), jnp.int32)]
    + [pltpu.VMEM((K, CP), jnp.float32)] * _NB
    + [pltpu.VMEM_SHARED((NP, CP), jnp.float32)]
    + [pltpu.SemaphoreType.DMA] * _NB
)


@functools.partial(pl.kernel, out_type=_L2_OUT, mesh=_MESH,
                   scratch_types=_L2_SCRATCH,
                   compiler_params=pltpu.CompilerParams(
                       use_tc_tiling_on_sc=False))
def _sc_layer2(p_hbm, src_hbm, dst_hbm, acc_out,
               src_v, dst_v, r0, r1, r2, r3, acc_sh, g0, g1, g2, g3):
  cid = lax.axis_index("c")
  sid = lax.axis_index("s")
  wid = sid * NC + cid

  def init_row(i, carry):
    _zero_fill(r0, i, CP)
    return carry

  lax.fori_loop(0, K, init_row, 0)
  base = sid * RPT
  for t in range(RPT // K):
    pltpu.sync_copy(r0, acc_sh.at[pl.ds(base + t * K, K)])
  if RPT % K:
    pltpu.sync_copy(r0.at[pl.ds(0, RPT % K)],
                    acc_sh.at[pl.ds(base + RPT - RPT % K, RPT % K)])
  plsc.subcore_barrier()

  pltpu.sync_copy(src_hbm.at[wid], src_v)
  pltpu.sync_copy(dst_hbm.at[wid], dst_v)

  _edge_pipeline(p_hbm, src_v, dst_v, (r0, r1, r2, r3),
                 (g0, g1, g2, g3), acc_sh, CH2)
  plsc.subcore_barrier()

  pltpu.sync_copy(acc_sh.at[pl.ds(base, RPT)],
                  acc_out.at[cid, pl.ds(base, RPT)])


RT1 = 1024


def _tc_layer0(x_lo, x_hi, wr1t, b1):
  def body(xlo_ref, xhi_ref, wr1_ref, b1_ref, xr_ref):
    x = jnp.concatenate([xlo_ref[...], xhi_ref[...]], axis=1)
    xr_ref[...] = x @ wr1_ref[...] + b1_ref[...]

  return pl.pallas_call(
      body,
      grid=(NP // RT1,),
      in_specs=[
          pl.BlockSpec((RT1, DH), lambda i: (i, 0)),
          pl.BlockSpec((RT1, DH), lambda i: (i, 0)),
          pl.BlockSpec((D, H), lambda i: (0, 0)),
          pl.BlockSpec((1, H), lambda i: (0, 0)),
      ],
      out_specs=pl.BlockSpec((RT1, H), lambda i: (i, 0)),
      out_shape=jax.ShapeDtypeStruct((NP, H), jnp.float32),
  )(x_lo, x_hi, wr1t, b1)


def _tc_layer1(acc_lo, acc_hi, cntp, xr, wl1t, wl2tp, wr2t):
  def body(lo_ref, hi_ref, cnt_ref, xr_ref, wl1_ref,
           wl2_ref, wr2_ref, p_ref, r_ref, cnt_out_ref):
    agg = jnp.concatenate([lo_ref[...], hi_ref[...]], axis=1)
    cnt_col = cnt_ref[0, :, 0:1] + cnt_ref[1, :, 0:1]
    cnt_out_ref[...] = cnt_col
    mean = agg / jnp.maximum(cnt_col, 1.0)
    h = jnp.maximum(mean @ wl1_ref[...] + xr_ref[...], 0.0)
    p_ref[...] = h @ wl2_ref[...]
    r_ref[...] = h @ wr2_ref[...]

  return pl.pallas_call(
      body,
      grid=(NP // RT1,),
      in_specs=[
          pl.BlockSpec((RT1, DH), lambda i: (i, 0)),
          pl.BlockSpec((RT1, DH), lambda i: (i, 0)),
          pl.BlockSpec((NC, RT1, 16), lambda i: (0, i, 0)),
          pl.BlockSpec((RT1, H), lambda i: (i, 0)),
          pl.BlockSpec((D, H), lambda i: (0, 0)),
          pl.BlockSpec((H, CP), lambda i: (0, 0)),
          pl.BlockSpec((H, C), lambda i: (0, 0)),
      ],
      out_specs=[
          pl.BlockSpec((RT1, CP), lambda i: (i, 0)),
          pl.BlockSpec((RT1, C), lambda i: (i, 0)),
          pl.BlockSpec((RT1, 1), lambda i: (i, 0)),
      ],
      out_shape=[
          jax.ShapeDtypeStruct((NP, CP), jnp.float32),
          jax.ShapeDtypeStruct((NP, C), jnp.float32),
          jax.ShapeDtypeStruct((NP, 1), jnp.float32),
      ],
  )(acc_lo, acc_hi, cntp, xr, wl1t, wl2tp, wr2t)


RT2 = 2000


def _tc_layer2(acc2, cnt, r_self, b2):
  def body(acc_ref, cnt_ref, r_ref, b2_ref, out_ref):
    agg = acc_ref[0, :, :C] + acc_ref[1, :, :C]
    cnt_col = cnt_ref[...]
    logits = agg / jnp.maximum(cnt_col, 1.0) + r_ref[...] + b2_ref[...]
    m = jnp.max(logits, axis=1, keepdims=True)
    lse = jnp.log(jnp.sum(jnp.exp(logits - m), axis=1, keepdims=True)) + m
    out_ref[...] = logits - lse

  return pl.pallas_call(
      body,
      grid=(N // RT2,),
      in_specs=[
          pl.BlockSpec((NC, RT2, CP), lambda i: (0, i, 0)),
          pl.BlockSpec((RT2, 1), lambda i: (i, 0)),
          pl.BlockSpec((RT2, C), lambda i: (i, 0)),
          pl.BlockSpec((1, C), lambda i: (0, 0)),
      ],
      out_specs=pl.BlockSpec((RT2, C), lambda i: (i, 0)),
      out_shape=jax.ShapeDtypeStruct((N, C), jnp.float32),
  )(acc2, cnt, r_self, b2)


def kernel(x, edge_index, Wl1, Wr1, b1, Wl2, Wr2, b2):
  src = edge_index[0].astype(jnp.int32)
  dst = edge_index[1].astype(jnp.int32)
  pad = N + jnp.arange(EP - E, dtype=jnp.int32) % (NP - N)
  src_p = jnp.concatenate([src, pad])
  dst_p = jnp.concatenate([dst, pad])
  src1 = src_p.reshape(NS, CH1, K)
  dst1 = dst_p.reshape(NS, CH1, K)
  src2 = src_p.reshape(NC * NS, CH2, K)
  dst2 = dst_p.reshape(NC * NS, CH2, K)
  x_lo = jnp.pad(x[:, :DH], ((0, NP - N), (0, 0)))
  x_hi = jnp.pad(x[:, DH:], ((0, NP - N), (0, 0)))

  wl1t = Wl1.T
  wr1t = Wr1.T
  b1r = b1.reshape(1, H)
  wl2tp = jnp.pad(Wl2.T, ((0, 0), (0, CP - C)))
  wr2t = Wr2.T
  b2r = b2.reshape(1, C)

  acc_lo, acc_hi, cntp = _sc_layer1(x_lo, x_hi, src1, dst1)
  xr = _tc_layer0(x_lo, x_hi, wr1t, b1r)
  p_pad, r_self, cnt_sum = _tc_layer1(acc_lo, acc_hi, cntp, xr,
                                      wl1t, wl2tp, wr2t)
  (acc2,) = _sc_layer2(p_pad, src2, dst2)
  return _tc_layer2(acc2, cnt_sum, r_self, b2r)


# speedup vs baseline: 1.0012x; 1.0012x over previous
import functools

import jax
import jax.numpy as jnp
from jax import lax
from jax.experimental import pallas as pl
from jax.experimental.pallas import tpu as pltpu
from jax.experimental.pallas import tpu_sc as plsc

N = 10000
D = 128
DH = D // 2
H = 256
C = 40
E = 320000

NP = 10240
CP = 48
NC = 2
NS = 16
K = 128
RPT = NP // NS

CH1 = 158
CH2 = 79
EP = NS * CH1 * K

_NB = 4

_MESH = plsc.VectorSubcoreMesh(core_axis_name="c", subcore_axis_name="s")


def _zero_fill(ref, i, width):
  for t in range(width // 16):
    ref[i, pl.ds(t * 16, 16)] = jnp.zeros((16,), jnp.float32)


def _edge_pipeline(tbl, src_v, dst_v, rows, gsems, acc_sh, num_chunks,
                   ones_v=None, cnt_sh=None, ones_parity=0):
  pltpu.async_copy(tbl.at[src_v.at[0]], rows[0], gsems[0])
  pltpu.async_copy(tbl.at[src_v.at[1]], rows[1], gsems[1])

  def group(g, carry):
    j0 = g * _NB
    for b in range(_NB):
      jj = j0 + b
      bn = (b + 2) % _NB

      @pl.when(jj < num_chunks)
      def _(jj=jj, b=b, bn=bn):
        pltpu.make_async_copy(tbl.at[src_v.at[jj]], rows[b], gsems[b]).wait()
        pltpu.sync_copy(rows[b], acc_sh.at[dst_v.at[jj]], add=True)
        if ones_v is not None:
          @pl.when(jj % 2 == ones_parity)
          def _():
            pltpu.sync_copy(ones_v, cnt_sh.at[dst_v.at[jj]], add=True)

        @pl.when(jj + 2 < num_chunks)
        def _():
          pltpu.async_copy(tbl.at[src_v.at[jj + 2]], rows[bn], gsems[bn])

    return carry

  lax.fori_loop(0, (num_chunks + _NB - 1) // _NB, group, 0)


_L1_OUT = [
    jax.ShapeDtypeStruct((NP, DH), jnp.float32),
    jax.ShapeDtypeStruct((NP, DH), jnp.float32),
    jax.ShapeDtypeStruct((NC, NP, 16), jnp.float32),
]
_L1_SCRATCH = (
    [pltpu.VMEM((CH1, K), jnp.int32),
     pltpu.VMEM((CH1, K), jnp.int32)]
    + [pltpu.VMEM((K, DH), jnp.float32)] * _NB
    + [pltpu.VMEM((K, 16), jnp.float32),
       pltpu.VMEM((K, 16), jnp.float32)]
    + [pltpu.VMEM_SHARED((NP, DH), jnp.float32),
       pltpu.VMEM_SHARED((NP, 16), jnp.float32)]
    + [pltpu.SemaphoreType.DMA] * _NB
)


@functools.partial(pl.kernel, out_type=_L1_OUT, mesh=_MESH,
                   scratch_types=_L1_SCRATCH,
                   compiler_params=pltpu.CompilerParams(
                       use_tc_tiling_on_sc=False))
def _sc_layer1(x_lo, x_hi, src_hbm, dst_hbm, lo_out, hi_out, cnt_out,
               src_v, dst_v, r0, r1, r2, r3, ones_v, zcnt_v, acc_sh, cnt_sh,
               g0, g1, g2, g3):
  cid = lax.axis_index("c")
  sid = lax.axis_index("s")
  rows = (r0, r1, r2, r3)
  gsems = (g0, g1, g2, g3)

  def init_row(i, carry):
    _zero_fill(r0, i, DH)
    ones_v[i, :] = jnp.ones((16,), jnp.float32)
    zcnt_v[i, :] = jnp.zeros((16,), jnp.float32)
    return carry

  lax.fori_loop(0, K, init_row, 0)
  base = sid * RPT
  for t in range(RPT // K):
    pltpu.sync_copy(r0, acc_sh.at[pl.ds(base + t * K, K)])
    pltpu.sync_copy(zcnt_v, cnt_sh.at[pl.ds(base + t * K, K)])
  plsc.subcore_barrier()

  pltpu.sync_copy(src_hbm.at[sid], src_v)
  pltpu.sync_copy(dst_hbm.at[sid], dst_v)

  @pl.when(cid == 0)
  def _():
    _edge_pipeline(x_lo, src_v, dst_v, rows, gsems, acc_sh, CH1,
                   ones_v, cnt_sh, ones_parity=0)

  @pl.when(cid == 1)
  def _():
    _edge_pipeline(x_hi, src_v, dst_v, rows, gsems, acc_sh, CH1,
                   ones_v, cnt_sh, ones_parity=1)

  plsc.subcore_barrier()

  @pl.when(cid == 0)
  def _():
    pltpu.sync_copy(acc_sh.at[pl.ds(base, RPT)], lo_out.at[pl.ds(base, RPT)])

  @pl.when(cid == 1)
  def _():
    pltpu.sync_copy(acc_sh.at[pl.ds(base, RPT)], hi_out.at[pl.ds(base, RPT)])

  pltpu.sync_copy(cnt_sh.at[pl.ds(base, RPT)],
                  cnt_out.at[cid, pl.ds(base, RPT)])


_L2_OUT = [jax.ShapeDtypeStruct((NC, NP, CP), jnp.float32)]
_L2_SCRATCH = (
    [pltpu.VMEM((CH2, K), jnp.int32),
     pltpu.VMEM((CH2, K), jnp.int32)]
    + [pltpu.VMEM((K, CP), jnp.float32)] * _NB
    + [pltpu.VMEM_SHARED((NP, CP), jnp.float32)]
    + [pltpu.SemaphoreType.DMA] * _NB
)


@functools.partial(pl.kernel, out_type=_L2_OUT, mesh=_MESH,
                   scratch_types=_L2_SCRATCH,
                   compiler_params=pltpu.CompilerParams(
                       use_tc_tiling_on_sc=False))
def _sc_layer2(p_hbm, src_hbm, dst_hbm, acc_out,
               src_v, dst_v, r0, r1, r2, r3, acc_sh, g0, g1, g2, g3):
  cid = lax.axis_index("c")
  sid = lax.axis_index("s")
  wid = sid * NC + cid

  def init_row(i, carry):
    _zero_fill(r0, i, CP)
    return carry

  lax.fori_loop(0, K, init_row, 0)
  base = sid * RPT
  for t in range(RPT // K):
    pltpu.sync_copy(r0, acc_sh.at[pl.ds(base + t * K, K)])
  if RPT % K:
    pltpu.sync_copy(r0.at[pl.ds(0, RPT % K)],
                    acc_sh.at[pl.ds(base + RPT - RPT % K, RPT % K)])
  plsc.subcore_barrier()

  pltpu.sync_copy(src_hbm.at[wid], src_v)
  pltpu.sync_copy(dst_hbm.at[wid], dst_v)

  _edge_pipeline(p_hbm, src_v, dst_v, (r0, r1, r2, r3),
                 (g0, g1, g2, g3), acc_sh, CH2)
  plsc.subcore_barrier()

  pltpu.sync_copy(acc_sh.at[pl.ds(base, RPT)],
                  acc_out.at[cid, pl.ds(base, RPT)])


RT1 = 1024


def _tc_layer0(x_lo, x_hi, wr1t, b1):
  def body(xlo_ref, xhi_ref, wr1_ref, b1_ref, xr_ref):
    x = jnp.concatenate([xlo_ref[...], xhi_ref[...]], axis=1)
    xr_ref[...] = x @ wr1_ref[...] + b1_ref[...]

  return pl.pallas_call(
      body,
      grid=(NP // RT1,),
      in_specs=[
          pl.BlockSpec((RT1, DH), lambda i: (i, 0)),
          pl.BlockSpec((RT1, DH), lambda i: (i, 0)),
          pl.BlockSpec((D, H), lambda i: (0, 0)),
          pl.BlockSpec((1, H), lambda i: (0, 0)),
      ],
      out_specs=pl.BlockSpec((RT1, H), lambda i: (i, 0)),
      out_shape=jax.ShapeDtypeStruct((NP, H), jnp.float32),
  )(x_lo, x_hi, wr1t, b1)


def _tc_layer1(acc_lo, acc_hi, cntp, xr, wl1t, wl2tp, wr2t):
  def body(lo_ref, hi_ref, cnt_ref, xr_ref, wl1_ref,
           wl2_ref, wr2_ref, p_ref, r_ref, cnt_out_ref):
    agg = jnp.concatenate([lo_ref[...], hi_ref[...]], axis=1)
    cnt_col = cnt_ref[0, :, 0:1] + cnt_ref[1, :, 0:1]
    cnt_out_ref[...] = cnt_col
    mean = agg / jnp.maximum(cnt_col, 1.0)
    h = jnp.maximum(mean @ wl1_ref[...] + xr_ref[...], 0.0)
    p_ref[...] = h @ wl2_ref[...]
    r_ref[...] = h @ wr2_ref[...]

  return pl.pallas_call(
      body,
      grid=(NP // RT1,),
      in_specs=[
          pl.BlockSpec((RT1, DH), lambda i: (i, 0)),
          pl.BlockSpec((RT1, DH), lambda i: (i, 0)),
          pl.BlockSpec((NC, RT1, 16), lambda i: (0, i, 0)),
          pl.BlockSpec((RT1, H), lambda i: (i, 0)),
          pl.BlockSpec((D, H), lambda i: (0, 0)),
          pl.BlockSpec((H, CP), lambda i: (0, 0)),
          pl.BlockSpec((H, C), lambda i: (0, 0)),
      ],
      out_specs=[
          pl.BlockSpec((RT1, CP), lambda i: (i, 0)),
          pl.BlockSpec((RT1, C), lambda i: (i, 0)),
          pl.BlockSpec((RT1, 1), lambda i: (i, 0)),
      ],
      out_shape=[
          jax.ShapeDtypeStruct((NP, CP), jnp.float32),
          jax.ShapeDtypeStruct((NP, C), jnp.float32),
          jax.ShapeDtypeStruct((NP, 1), jnp.float32),
      ],
  )(acc_lo, acc_hi, cntp, xr, wl1t, wl2tp, wr2t)


RT2 = 2000


def _tc_layer2(acc2, cnt, r_self, b2):
  def body(acc_ref, cnt_ref, r_ref, b2_ref, out_ref):
    agg = acc_ref[0, :, :C] + acc_ref[1, :, :C]
    cnt_col = cnt_ref[...]
    logits = agg / jnp.maximum(cnt_col, 1.0) + r_ref[...] + b2_ref[...]
    m = jnp.max(logits, axis=1, keepdims=True)
    lse = jnp.log(jnp.sum(jnp.exp(logits - m), axis=1, keepdims=True)) + m
    out_ref[...] = logits - lse

  return pl.pallas_call(
      body,
      grid=(N // RT2,),
      in_specs=[
          pl.BlockSpec((NC, RT2, CP), lambda i: (0, i, 0)),
          pl.BlockSpec((RT2, 1), lambda i: (i, 0)),
          pl.BlockSpec((RT2, C), lambda i: (i, 0)),
          pl.BlockSpec((1, C), lambda i: (0, 0)),
      ],
      out_specs=pl.BlockSpec((RT2, C), lambda i: (i, 0)),
      out_shape=jax.ShapeDtypeStruct((N, C), jnp.float32),
  )(acc2, cnt, r_self, b2)


def kernel(x, edge_index, Wl1, Wr1, b1, Wl2, Wr2, b2):
  src = edge_index[0].astype(jnp.int32)
  dst = edge_index[1].astype(jnp.int32)
  pad = N + jnp.arange(EP - E, dtype=jnp.int32) % (NP - N)
  src_p = jnp.concatenate([src, pad])
  dst_p = jnp.concatenate([dst, pad])
  src1 = src_p.reshape(NS, CH1, K)
  dst1 = dst_p.reshape(NS, CH1, K)
  src2 = src_p.reshape(NC * NS, CH2, K)
  dst2 = dst_p.reshape(NC * NS, CH2, K)
  x_lo = jnp.pad(x[:, :DH], ((0, NP - N), (0, 0)))
  x_hi = jnp.pad(x[:, DH:], ((0, NP - N), (0, 0)))

  wl1t = Wl1.T
  wr1t = Wr1.T
  b1r = b1.reshape(1, H)
  wl2tp = jnp.pad(Wl2.T, ((0, 0), (0, CP - C)))
  wr2t = Wr2.T
  b2r = b2.reshape(1, C)

  xr = _tc_layer0(x_lo, x_hi, wr1t, b1r)
  acc_lo, acc_hi, cntp = _sc_layer1(x_lo, x_hi, src1, dst1)
  p_pad, r_self, cnt_sum = _tc_layer1(acc_lo, acc_hi, cntp, xr,
                                      wl1t, wl2tp, wr2t)
  (acc2,) = _sc_layer2(p_pad, src2, dst2)
  return _tc_layer2(acc2, cnt_sum, r_self, b2r)


# speedup vs baseline: 1.0578x; 1.0565x over previous
import functools

import jax
import jax.numpy as jnp
from jax import lax
from jax.experimental import pallas as pl
from jax.experimental.pallas import tpu as pltpu
from jax.experimental.pallas import tpu_sc as plsc

N = 10000
D = 128
DH = D // 2
H = 256
C = 40
E = 320000

NP = 10240
CP = 48
NC = 2
NS = 16
K = 128
RPT = NP // NS

CH1 = 158
CH2 = 79
EP = NS * CH1 * K

_NB = 4

_MESH = plsc.VectorSubcoreMesh(core_axis_name="c", subcore_axis_name="s")


def _zero_fill(ref, i, width):
  for t in range(width // 16):
    ref[i, pl.ds(t * 16, 16)] = jnp.zeros((16,), jnp.float32)


def _edge_pipeline(tbl, src_v, dst_v, rows, gsems, acc_sh, num_chunks,
                   ones_v=None, cnt_sh=None, ones_parity=0):
  pltpu.async_copy(tbl.at[src_v.at[0]], rows[0], gsems[0])
  pltpu.async_copy(tbl.at[src_v.at[1]], rows[1], gsems[1])

  def group(g, carry):
    j0 = g * _NB
    for b in range(_NB):
      jj = j0 + b
      bn = (b + 2) % _NB

      @pl.when(jj < num_chunks)
      def _(jj=jj, b=b, bn=bn):
        pltpu.make_async_copy(tbl.at[src_v.at[jj]], rows[b], gsems[b]).wait()
        pltpu.sync_copy(rows[b], acc_sh.at[dst_v.at[jj]], add=True)
        if ones_v is not None:
          @pl.when(jj % 2 == ones_parity)
          def _():
            pltpu.sync_copy(ones_v, cnt_sh.at[dst_v.at[jj]], add=True)

        @pl.when(jj + 2 < num_chunks)
        def _():
          pltpu.async_copy(tbl.at[src_v.at[jj + 2]], rows[bn], gsems[bn])

    return carry

  lax.fori_loop(0, (num_chunks + _NB - 1) // _NB, group, 0)


_L1_OUT = [
    jax.ShapeDtypeStruct((NP, D), jnp.float32),
    jax.ShapeDtypeStruct((NC, NP, 16), jnp.float32),
]
_L1_SCRATCH = (
    [pltpu.VMEM((CH1, K), jnp.int32),
     pltpu.VMEM((CH1, K), jnp.int32)]
    + [pltpu.VMEM((K, DH), jnp.float32)] * _NB
    + [pltpu.VMEM((K, 16), jnp.float32),
       pltpu.VMEM((K, 16), jnp.float32)]
    + [pltpu.VMEM_SHARED((NP, DH), jnp.float32),
       pltpu.VMEM_SHARED((NP, 16), jnp.float32)]
    + [pltpu.SemaphoreType.DMA] * _NB
)


@functools.partial(pl.kernel, out_type=_L1_OUT, mesh=_MESH,
                   scratch_types=_L1_SCRATCH,
                   compiler_params=pltpu.CompilerParams(
                       use_tc_tiling_on_sc=False))
def _sc_layer1(x_lo, x_hi, src_hbm, dst_hbm, acc_out, cnt_out,
               src_v, dst_v, r0, r1, r2, r3, ones_v, zcnt_v, acc_sh, cnt_sh,
               g0, g1, g2, g3):
  cid = lax.axis_index("c")
  sid = lax.axis_index("s")
  rows = (r0, r1, r2, r3)
  gsems = (g0, g1, g2, g3)

  def init_row(i, carry):
    _zero_fill(r0, i, DH)
    ones_v[i, :] = jnp.ones((16,), jnp.float32)
    zcnt_v[i, :] = jnp.zeros((16,), jnp.float32)
    return carry

  lax.fori_loop(0, K, init_row, 0)
  base = sid * RPT
  for t in range(RPT // K):
    pltpu.sync_copy(r0, acc_sh.at[pl.ds(base + t * K, K)])
    pltpu.sync_copy(zcnt_v, cnt_sh.at[pl.ds(base + t * K, K)])
  plsc.subcore_barrier()

  pltpu.sync_copy(src_hbm.at[sid], src_v)
  pltpu.sync_copy(dst_hbm.at[sid], dst_v)

  @pl.when(cid == 0)
  def _():
    _edge_pipeline(x_lo, src_v, dst_v, rows, gsems, acc_sh, CH1,
                   ones_v, cnt_sh, ones_parity=0)

  @pl.when(cid == 1)
  def _():
    _edge_pipeline(x_hi, src_v, dst_v, rows, gsems, acc_sh, CH1,
                   ones_v, cnt_sh, ones_parity=1)

  plsc.subcore_barrier()

  @pl.when(cid == 0)
  def _():
    pltpu.sync_copy(acc_sh.at[pl.ds(base, RPT)],
                    acc_out.at[pl.ds(base, RPT), pl.ds(0, DH)])

  @pl.when(cid == 1)
  def _():
    pltpu.sync_copy(acc_sh.at[pl.ds(base, RPT)],
                    acc_out.at[pl.ds(base, RPT), pl.ds(DH, DH)])

  pltpu.sync_copy(cnt_sh.at[pl.ds(base, RPT)],
                  cnt_out.at[cid, pl.ds(base, RPT)])


_L2_OUT = [jax.ShapeDtypeStruct((NP, D), jnp.float32)]
_L2_SCRATCH = (
    [pltpu.VMEM((CH2, K), jnp.int32),
     pltpu.VMEM((CH2, K), jnp.int32)]
    + [pltpu.VMEM((K, CP), jnp.float32)] * _NB
    + [pltpu.VMEM_SHARED((NP, CP), jnp.float32)]
    + [pltpu.SemaphoreType.DMA] * _NB
)


@functools.partial(pl.kernel, out_type=_L2_OUT, mesh=_MESH,
                   scratch_types=_L2_SCRATCH,
                   compiler_params=pltpu.CompilerParams(
                       use_tc_tiling_on_sc=False))
def _sc_layer2(p_hbm, src_hbm, dst_hbm, acc_out,
               src_v, dst_v, r0, r1, r2, r3, acc_sh, g0, g1, g2, g3):
  cid = lax.axis_index("c")
  sid = lax.axis_index("s")
  wid = sid * NC + cid

  def init_row(i, carry):
    _zero_fill(r0, i, CP)
    return carry

  lax.fori_loop(0, K, init_row, 0)
  base = sid * RPT
  for t in range(RPT // K):
    pltpu.sync_copy(r0, acc_sh.at[pl.ds(base + t * K, K)])
  if RPT % K:
    pltpu.sync_copy(r0.at[pl.ds(0, RPT % K)],
                    acc_sh.at[pl.ds(base + RPT - RPT % K, RPT % K)])
  plsc.subcore_barrier()

  pltpu.sync_copy(src_hbm.at[wid], src_v)
  pltpu.sync_copy(dst_hbm.at[wid], dst_v)

  _edge_pipeline(p_hbm, src_v, dst_v, (r0, r1, r2, r3),
                 (g0, g1, g2, g3), acc_sh, CH2)
  plsc.subcore_barrier()

  @pl.when(cid == 0)
  def _():
    pltpu.sync_copy(acc_sh.at[pl.ds(base, RPT)],
                    acc_out.at[pl.ds(base, RPT), pl.ds(0, CP)])

  @pl.when(cid == 1)
  def _():
    pltpu.sync_copy(acc_sh.at[pl.ds(base, RPT)],
                    acc_out.at[pl.ds(base, RPT), pl.ds(DH, CP)])


RT1 = 1024


def _tc_layer1(acc, cntp, x_lo, x_hi, wl1t, wr1t, b1, wl2tp, wr2t):
  def body(acc_ref, cnt_ref, xlo_ref, xhi_ref, wl1_ref, wr1_ref, b1_ref,
           wl2_ref, wr2_ref, p_ref, r_ref, cnt_out_ref):
    x = jnp.concatenate([xlo_ref[...], xhi_ref[...]], axis=1)
    cnt_col = cnt_ref[0, :, 0:1] + cnt_ref[1, :, 0:1]
    cnt_out_ref[...] = cnt_col
    mean = acc_ref[...] / jnp.maximum(cnt_col, 1.0)
    h = mean @ wl1_ref[...] + x @ wr1_ref[...] + b1_ref[...]
    h = jnp.maximum(h, 0.0)
    p_ref[...] = h @ wl2_ref[...]
    r_ref[...] = h @ wr2_ref[...]

  return pl.pallas_call(
      body,
      grid=(NP // RT1,),
      in_specs=[
          pl.BlockSpec((RT1, D), lambda i: (i, 0)),
          pl.BlockSpec((NC, RT1, 16), lambda i: (0, i, 0)),
          pl.BlockSpec((RT1, DH), lambda i: (i, 0)),
          pl.BlockSpec((RT1, DH), lambda i: (i, 0)),
          pl.BlockSpec((D, H), lambda i: (0, 0)),
          pl.BlockSpec((D, H), lambda i: (0, 0)),
          pl.BlockSpec((1, H), lambda i: (0, 0)),
          pl.BlockSpec((H, CP), lambda i: (0, 0)),
          pl.BlockSpec((H, C), lambda i: (0, 0)),
      ],
      out_specs=[
          pl.BlockSpec((RT1, CP), lambda i: (i, 0)),
          pl.BlockSpec((RT1, C), lambda i: (i, 0)),
          pl.BlockSpec((RT1, 1), lambda i: (i, 0)),
      ],
      out_shape=[
          jax.ShapeDtypeStruct((NP, CP), jnp.float32),
          jax.ShapeDtypeStruct((NP, C), jnp.float32),
          jax.ShapeDtypeStruct((NP, 1), jnp.float32),
      ],
  )(acc, cntp, x_lo, x_hi, wl1t, wr1t, b1, wl2tp, wr2t)


RT2 = 2000


def _tc_layer2(acc2, cnt, r_self, b2):
  def body(acc_ref, cnt_ref, r_ref, b2_ref, out_ref):
    agg = acc_ref[:, :C] + acc_ref[:, DH:DH + C]
    cnt_col = cnt_ref[...]
    logits = agg / jnp.maximum(cnt_col, 1.0) + r_ref[...] + b2_ref[...]
    m = jnp.max(logits, axis=1, keepdims=True)
    lse = jnp.log(jnp.sum(jnp.exp(logits - m), axis=1, keepdims=True)) + m
    out_ref[...] = logits - lse

  return pl.pallas_call(
      body,
      grid=(N // RT2,),
      in_specs=[
          pl.BlockSpec((RT2, D), lambda i: (i, 0)),
          pl.BlockSpec((RT2, 1), lambda i: (i, 0)),
          pl.BlockSpec((RT2, C), lambda i: (i, 0)),
          pl.BlockSpec((1, C), lambda i: (0, 0)),
      ],
      out_specs=pl.BlockSpec((RT2, C), lambda i: (i, 0)),
      out_shape=jax.ShapeDtypeStruct((N, C), jnp.float32),
  )(acc2, cnt, r_self, b2)


def kernel(x, edge_index, Wl1, Wr1, b1, Wl2, Wr2, b2):
  src = edge_index[0].astype(jnp.int32)
  dst = edge_index[1].astype(jnp.int32)
  pad = N + jnp.arange(EP - E, dtype=jnp.int32) % (NP - N)
  src_p = jnp.concatenate([src, pad])
  dst_p = jnp.concatenate([dst, pad])
  src1 = src_p.reshape(NS, CH1, K)
  dst1 = dst_p.reshape(NS, CH1, K)
  src2 = src_p.reshape(NC * NS, CH2, K)
  dst2 = dst_p.reshape(NC * NS, CH2, K)
  x_lo = jnp.pad(x[:, :DH], ((0, NP - N), (0, 0)))
  x_hi = jnp.pad(x[:, DH:], ((0, NP - N), (0, 0)))

  wl1t = Wl1.T
  wr1t = Wr1.T
  b1r = b1.reshape(1, H)
  wl2tp = jnp.pad(Wl2.T, ((0, 0), (0, CP - C)))
  wr2t = Wr2.T
  b2r = b2.reshape(1, C)

  acc1, cntp = _sc_layer1(x_lo, x_hi, src1, dst1)
  p_pad, r_self, cnt_sum = _tc_layer1(acc1, cntp, x_lo, x_hi, wl1t, wr1t,
                                      b1r, wl2tp, wr2t)
  (acc2,) = _sc_layer2(p_pad, src2, dst2)
  return _tc_layer2(acc2, cnt_sum, r_self, b2r)


# speedup vs baseline: 1.0763x; 1.0175x over previous
import functools

import jax
import jax.numpy as jnp
from jax import lax
from jax.experimental import pallas as pl
from jax.experimental.pallas import tpu as pltpu
from jax.experimental.pallas import tpu_sc as plsc

N = 10000
D = 128
DH = D // 2
H = 256
C = 40
E = 320000

NP = 10240
CP = 48
NC = 2
NS = 16
K = 128
RPT = NP // NS

CH1 = 158
CH2 = 79
EP = NS * CH1 * K

_NB = 4

_MESH = plsc.VectorSubcoreMesh(core_axis_name="c", subcore_axis_name="s")


def _zero_fill(ref, i, width):
  for t in range(width // 16):
    ref[i, pl.ds(t * 16, 16)] = jnp.zeros((16,), jnp.float32)


def _edge_pipeline(tbl, src_v, dst_v, rows, gsems, acc_sh, num_chunks,
                   ones_v=None, cnt_sh=None, ones_parity=0):
  pltpu.async_copy(tbl.at[src_v.at[0]], rows[0], gsems[0])
  pltpu.async_copy(tbl.at[src_v.at[1]], rows[1], gsems[1])

  def group(g, carry):
    j0 = g * _NB
    for b in range(_NB):
      jj = j0 + b
      bn = (b + 2) % _NB

      @pl.when(jj < num_chunks)
      def _(jj=jj, b=b, bn=bn):
        pltpu.make_async_copy(tbl.at[src_v.at[jj]], rows[b], gsems[b]).wait()
        pltpu.sync_copy(rows[b], acc_sh.at[dst_v.at[jj]], add=True)
        if ones_v is not None:
          @pl.when(jj % 2 == ones_parity)
          def _():
            pltpu.sync_copy(ones_v, cnt_sh.at[dst_v.at[jj]], add=True)

        @pl.when(jj + 2 < num_chunks)
        def _():
          pltpu.async_copy(tbl.at[src_v.at[jj + 2]], rows[bn], gsems[bn])

    return carry

  lax.fori_loop(0, (num_chunks + _NB - 1) // _NB, group, 0)


_L1_OUT = [
    jax.ShapeDtypeStruct((NP, D), jnp.float32),
    jax.ShapeDtypeStruct((NC, NP, 16), jnp.float32),
]
_L1_SCRATCH = (
    [pltpu.VMEM((CH1, K), jnp.int32),
     pltpu.VMEM((CH1, K), jnp.int32),
     pltpu.SemaphoreType.DMA]
    + [pltpu.VMEM((K, DH), jnp.float32)] * _NB
    + [pltpu.VMEM((K, 16), jnp.float32),
       pltpu.VMEM((K, 16), jnp.float32)]
    + [pltpu.VMEM_SHARED((NP, DH), jnp.float32),
       pltpu.VMEM_SHARED((NP, 16), jnp.float32)]
    + [pltpu.SemaphoreType.DMA] * _NB
)


@functools.partial(pl.kernel, out_type=_L1_OUT, mesh=_MESH,
                   scratch_types=_L1_SCRATCH,
                   compiler_params=pltpu.CompilerParams(
                       use_tc_tiling_on_sc=False))
def _sc_layer1(x_lo, x_hi, src_hbm, dst_hbm, acc_out, cnt_out,
               src_v, dst_v, isem, r0, r1, r2, r3, ones_v, zcnt_v, acc_sh,
               cnt_sh, g0, g1, g2, g3):
  cid = lax.axis_index("c")
  sid = lax.axis_index("s")
  rows = (r0, r1, r2, r3)
  gsems = (g0, g1, g2, g3)
  src_slab = src_hbm.at[pl.ds(sid * CH1, CH1)]
  dst_slab = dst_hbm.at[pl.ds(sid * CH1, CH1)]
  pltpu.async_copy(src_slab, src_v, isem)
  pltpu.async_copy(dst_slab, dst_v, isem)

  def init_row(i, carry):
    _zero_fill(r0, i, DH)
    ones_v[i, :] = jnp.ones((16,), jnp.float32)
    zcnt_v[i, :] = jnp.zeros((16,), jnp.float32)
    return carry

  lax.fori_loop(0, K, init_row, 0)
  base = sid * RPT
  for t in range(RPT // K):
    pltpu.sync_copy(r0, acc_sh.at[pl.ds(base + t * K, K)])
    pltpu.sync_copy(zcnt_v, cnt_sh.at[pl.ds(base + t * K, K)])
  plsc.subcore_barrier()

  pltpu.make_async_copy(src_slab, src_v, isem).wait()
  pltpu.make_async_copy(dst_slab, dst_v, isem).wait()

  @pl.when(cid == 0)
  def _():
    _edge_pipeline(x_lo, src_v, dst_v, rows, gsems, acc_sh, CH1,
                   ones_v, cnt_sh, ones_parity=0)

  @pl.when(cid == 1)
  def _():
    _edge_pipeline(x_hi, src_v, dst_v, rows, gsems, acc_sh, CH1,
                   ones_v, cnt_sh, ones_parity=1)

  plsc.subcore_barrier()

  @pl.when(cid == 0)
  def _():
    pltpu.sync_copy(acc_sh.at[pl.ds(base, RPT)],
                    acc_out.at[pl.ds(base, RPT), pl.ds(0, DH)])

  @pl.when(cid == 1)
  def _():
    pltpu.sync_copy(acc_sh.at[pl.ds(base, RPT)],
                    acc_out.at[pl.ds(base, RPT), pl.ds(DH, DH)])

  pltpu.sync_copy(cnt_sh.at[pl.ds(base, RPT)],
                  cnt_out.at[cid, pl.ds(base, RPT)])


_L2_OUT = [jax.ShapeDtypeStruct((NP, D), jnp.float32)]
_L2_SCRATCH = (
    [pltpu.VMEM((CH2, K), jnp.int32),
     pltpu.VMEM((CH2, K), jnp.int32),
     pltpu.SemaphoreType.DMA]
    + [pltpu.VMEM((K, CP), jnp.float32)] * _NB
    + [pltpu.VMEM_SHARED((NP, CP), jnp.float32)]
    + [pltpu.SemaphoreType.DMA] * _NB
)


@functools.partial(pl.kernel, out_type=_L2_OUT, mesh=_MESH,
                   scratch_types=_L2_SCRATCH,
                   compiler_params=pltpu.CompilerParams(
                       use_tc_tiling_on_sc=False))
def _sc_layer2(p_hbm, src_hbm, dst_hbm, acc_out,
               src_v, dst_v, isem, r0, r1, r2, r3, acc_sh, g0, g1, g2, g3):
  cid = lax.axis_index("c")
  sid = lax.axis_index("s")
  wid = sid * NC + cid
  src_slab = src_hbm.at[pl.ds(wid * CH2, CH2)]
  dst_slab = dst_hbm.at[pl.ds(wid * CH2, CH2)]
  pltpu.async_copy(src_slab, src_v, isem)
  pltpu.async_copy(dst_slab, dst_v, isem)

  def init_row(i, carry):
    _zero_fill(r0, i, CP)
    return carry

  lax.fori_loop(0, K, init_row, 0)
  base = sid * RPT
  for t in range(RPT // K):
    pltpu.sync_copy(r0, acc_sh.at[pl.ds(base + t * K, K)])
  if RPT % K:
    pltpu.sync_copy(r0.at[pl.ds(0, RPT % K)],
                    acc_sh.at[pl.ds(base + RPT - RPT % K, RPT % K)])
  plsc.subcore_barrier()

  pltpu.make_async_copy(src_slab, src_v, isem).wait()
  pltpu.make_async_copy(dst_slab, dst_v, isem).wait()

  _edge_pipeline(p_hbm, src_v, dst_v, (r0, r1, r2, r3),
                 (g0, g1, g2, g3), acc_sh, CH2)
  plsc.subcore_barrier()

  @pl.when(cid == 0)
  def _():
    pltpu.sync_copy(acc_sh.at[pl.ds(base, RPT)],
                    acc_out.at[pl.ds(base, RPT), pl.ds(0, CP)])

  @pl.when(cid == 1)
  def _():
    pltpu.sync_copy(acc_sh.at[pl.ds(base, RPT)],
                    acc_out.at[pl.ds(base, RPT), pl.ds(DH, CP)])


RT1 = 1024


def _tc_layer1(acc, cntp, x_lo, x_hi, wl1t, wr1t, b1, wl2tp, wr2t):
  def body(acc_ref, cnt_ref, xlo_ref, xhi_ref, wl1_ref, wr1_ref, b1_ref,
           wl2_ref, wr2_ref, p_ref, r_ref, cnt_out_ref):
    x = jnp.concatenate([xlo_ref[...], xhi_ref[...]], axis=1)
    cnt_col = cnt_ref[0, :, 0:1] + cnt_ref[1, :, 0:1]
    cnt_out_ref[...] = cnt_col
    mean = acc_ref[...] / jnp.maximum(cnt_col, 1.0)
    h = mean @ wl1_ref[...] + x @ wr1_ref[...] + b1_ref[...]
    h = jnp.maximum(h, 0.0)
    p_ref[...] = h @ wl2_ref[...]
    r_ref[...] = h @ wr2_ref[...]

  return pl.pallas_call(
      body,
      grid=(NP // RT1,),
      in_specs=[
          pl.BlockSpec((RT1, D), lambda i: (i, 0)),
          pl.BlockSpec((NC, RT1, 16), lambda i: (0, i, 0)),
          pl.BlockSpec((RT1, DH), lambda i: (i, 0)),
          pl.BlockSpec((RT1, DH), lambda i: (i, 0)),
          pl.BlockSpec((D, H), lambda i: (0, 0)),
          pl.BlockSpec((D, H), lambda i: (0, 0)),
          pl.BlockSpec((1, H), lambda i: (0, 0)),
          pl.BlockSpec((H, CP), lambda i: (0, 0)),
          pl.BlockSpec((H, C), lambda i: (0, 0)),
      ],
      out_specs=[
          pl.BlockSpec((RT1, CP), lambda i: (i, 0)),
          pl.BlockSpec((RT1, C), lambda i: (i, 0)),
          pl.BlockSpec((RT1, 1), lambda i: (i, 0)),
      ],
      out_shape=[
          jax.ShapeDtypeStruct((NP, CP), jnp.float32),
          jax.ShapeDtypeStruct((NP, C), jnp.float32),
          jax.ShapeDtypeStruct((NP, 1), jnp.float32),
      ],
  )(acc, cntp, x_lo, x_hi, wl1t, wr1t, b1, wl2tp, wr2t)


RT2 = 2000


def _tc_layer2(acc2, cnt, r_self, b2):
  def body(acc_ref, cnt_ref, r_ref, b2_ref, out_ref):
    agg = acc_ref[:, :C] + acc_ref[:, DH:DH + C]
    cnt_col = cnt_ref[...]
    logits = agg / jnp.maximum(cnt_col, 1.0) + r_ref[...] + b2_ref[...]
    m = jnp.max(logits, axis=1, keepdims=True)
    lse = jnp.log(jnp.sum(jnp.exp(logits - m), axis=1, keepdims=True)) + m
    out_ref[...] = logits - lse

  return pl.pallas_call(
      body,
      grid=(N // RT2,),
      in_specs=[
          pl.BlockSpec((RT2, D), lambda i: (i, 0)),
          pl.BlockSpec((RT2, 1), lambda i: (i, 0)),
          pl.BlockSpec((RT2, C), lambda i: (i, 0)),
          pl.BlockSpec((1, C), lambda i: (0, 0)),
      ],
      out_specs=pl.BlockSpec((RT2, C), lambda i: (i, 0)),
      out_shape=jax.ShapeDtypeStruct((N, C), jnp.float32),
  )(acc2, cnt, r_self, b2)


def kernel(x, edge_index, Wl1, Wr1, b1, Wl2, Wr2, b2):
  src = edge_index[0].astype(jnp.int32)
  dst = edge_index[1].astype(jnp.int32)
  pad = N + jnp.arange(EP - E, dtype=jnp.int32) % (NP - N)
  src_p = jnp.concatenate([src, pad]).reshape(EP // K, K)
  dst_p = jnp.concatenate([dst, pad]).reshape(EP // K, K)
  x_lo = jnp.pad(x[:, :DH], ((0, NP - N), (0, 0)))
  x_hi = jnp.pad(x[:, DH:], ((0, NP - N), (0, 0)))

  wl1t = Wl1.T
  wr1t = Wr1.T
  b1r = b1.reshape(1, H)
  wl2tp = jnp.pad(Wl2.T, ((0, 0), (0, CP - C)))
  wr2t = Wr2.T
  b2r = b2.reshape(1, C)

  acc1, cntp = _sc_layer1(x_lo, x_hi, src_p, dst_p)
  p_pad, r_self, cnt_sum = _tc_layer1(acc1, cntp, x_lo, x_hi, wl1t, wr1t,
                                      b1r, wl2tp, wr2t)
  (acc2,) = _sc_layer2(p_pad, src_p, dst_p)
  return _tc_layer2(acc2, cnt_sum, r_self, b2r)
